# baseline (speedup 1.0000x reference)
import jax
import jax.numpy as jnp
from jax import lax
from jax.experimental import pallas as pl
from jax.experimental.pallas import tpu as pltpu
from jax.experimental.pallas import tpu_sc as plsc

BATCH = 16384
X_DIM = 1000
XC_DIM = 64
LANES = 16

NUM_CORES = 2
NUM_SUBCORES = 16
NUM_WORKERS = NUM_CORES * NUM_SUBCORES
ROWS_PER_WORKER = BATCH // NUM_WORKERS
NUM_CHUNKS = ROWS_PER_WORKER // LANES


def _tec_body(x_hbm, table_hbm, out_hbm, xbuf, idxbuf, rowsbuf, sem):
    wid = lax.axis_index("s") * NUM_CORES + lax.axis_index("c")
    base = wid * ROWS_PER_WORKER
    lane = lax.iota(jnp.int32, LANES)

    def chunk_body(g, _):
        row0 = base + g * LANES
        pltpu.sync_copy(x_hbm.at[pl.ds(row0, LANES)], xbuf)

        def col_body(c, carry):
            m, idx = carry
            cvec = jnp.broadcast_to(c, (LANES,))
            v = plsc.load_gather(xbuf, [lane, cvec])
            cond = v > m
            m = jnp.where(cond, v, m)
            idx = jnp.where(cond, cvec, idx)
            return m, idx

        init = (
            jnp.full((LANES,), -jnp.inf, jnp.float32),
            jnp.zeros((LANES,), jnp.int32),
        )
        idx = lax.fori_loop(0, X_DIM, col_body, init, unroll=8)[1]
        idxbuf[...] = idx

        pltpu.async_copy(table_hbm.at[idxbuf], rowsbuf, sem).wait()
        pltpu.sync_copy(rowsbuf, out_hbm.at[pl.ds(row0, LANES)])
        return _

    lax.fori_loop(0, NUM_CHUNKS, chunk_body, None)


@jax.jit
def kernel(x, twohot_table):
    mesh = plsc.VectorSubcoreMesh(core_axis_name="c", subcore_axis_name="s")
    run = pl.kernel(
        _tec_body,
        out_type=jax.ShapeDtypeStruct((BATCH, XC_DIM), jnp.float32),
        mesh=mesh,
        scratch_types=[
            pltpu.VMEM((LANES, X_DIM), jnp.float32),
            pltpu.VMEM((LANES,), jnp.int32),
            pltpu.VMEM((LANES, XC_DIM), jnp.float32),
            pltpu.SemaphoreType.DMA,
        ],
        compiler_params=pltpu.CompilerParams(
            use_tc_tiling_on_sc=False, needs_layout_passes=False
        ),
    )
    return run(x, twohot_table)


# speedup vs baseline: 1.1987x; 1.1987x over previous
import jax
import jax.numpy as jnp
from jax import lax
from jax.experimental import pallas as pl
from jax.experimental.pallas import tpu as pltpu
from jax.experimental.pallas import tpu_sc as plsc

BATCH = 16384
X_DIM = 1000
XC_DIM = 64
LANES = 16

NUM_CORES = 2
NUM_SUBCORES = 16
NUM_WORKERS = NUM_CORES * NUM_SUBCORES
ROWS_PER_WORKER = BATCH // NUM_WORKERS
NUM_CHUNKS = ROWS_PER_WORKER // LANES

NUM_STRANDS = 4
COLS_PER_STRAND = X_DIM // NUM_STRANDS


def _argmax16(xbuf, lane):
    def col_body(c, carry):
        ms, idxs = carry
        cvec = jnp.broadcast_to(c, (LANES,))
        new_ms = []
        new_idxs = []
        for k in range(NUM_STRANDS):
            ck = cvec + jnp.int32(k * COLS_PER_STRAND)
            v = plsc.load_gather(xbuf, [lane, ck])
            cond = v > ms[k]
            new_ms.append(jnp.where(cond, v, ms[k]))
            new_idxs.append(jnp.where(cond, ck, idxs[k]))
        return tuple(new_ms), tuple(new_idxs)

    init = (
        tuple(jnp.full((LANES,), -jnp.inf, jnp.float32) for _ in range(NUM_STRANDS)),
        tuple(jnp.zeros((LANES,), jnp.int32) for _ in range(NUM_STRANDS)),
    )
    ms, idxs = lax.fori_loop(0, COLS_PER_STRAND, col_body, init, unroll=4)

    m, idx = ms[0], idxs[0]
    for k in range(1, NUM_STRANDS):
        cond = ms[k] > m
        m = jnp.where(cond, ms[k], m)
        idx = jnp.where(cond, idxs[k], idx)
    return idx


def _tec_body(x_hbm, table_hbm, out_hbm, xbuf0, xbuf1, idxbuf, rowsbuf, sem0, sem1, semg):
    wid = lax.axis_index("s") * NUM_CORES + lax.axis_index("c")
    base = wid * ROWS_PER_WORKER
    lane = lax.iota(jnp.int32, LANES)

    def x_slice(g):
        return x_hbm.at[pl.ds(base + g * LANES, LANES)]

    def finish_chunk(g, xbuf):
        idxbuf[...] = _argmax16(xbuf, lane)
        pltpu.async_copy(table_hbm.at[idxbuf], rowsbuf, semg).wait()
        pltpu.sync_copy(rowsbuf, out_hbm.at[pl.ds(base + g * LANES, LANES)])

    pltpu.async_copy(x_slice(0), xbuf0, sem0)

    def pair_body(h, _):
        g0 = 2 * h
        pltpu.async_copy(x_slice(g0 + 1), xbuf1, sem1)
        pltpu.make_async_copy(x_slice(g0), xbuf0, sem0).wait()
        finish_chunk(g0, xbuf0)

        @pl.when(h + 1 < NUM_CHUNKS // 2)
        def _prefetch():
            pltpu.async_copy(x_slice(g0 + 2), xbuf0, sem0)

        pltpu.make_async_copy(x_slice(g0 + 1), xbuf1, sem1).wait()
        finish_chunk(g0 + 1, xbuf1)
        return _

    lax.fori_loop(0, NUM_CHUNKS // 2, pair_body, None)


@jax.jit
def kernel(x, twohot_table):
    mesh = plsc.VectorSubcoreMesh(core_axis_name="c", subcore_axis_name="s")
    run = pl.kernel(
        _tec_body,
        out_type=jax.ShapeDtypeStruct((BATCH, XC_DIM), jnp.float32),
        mesh=mesh,
        scratch_types=[
            pltpu.VMEM((LANES, X_DIM), jnp.float32),
            pltpu.VMEM((LANES, X_DIM), jnp.float32),
            pltpu.VMEM((LANES,), jnp.int32),
            pltpu.VMEM((LANES, XC_DIM), jnp.float32),
            pltpu.SemaphoreType.DMA,
            pltpu.SemaphoreType.DMA,
            pltpu.SemaphoreType.DMA,
        ],
        compiler_params=pltpu.CompilerParams(
            use_tc_tiling_on_sc=False, needs_layout_passes=False
        ),
    )
    return run(x, twohot_table)
